# baseline (speedup 1.0000x reference)
import functools

import jax
import jax.numpy as jnp
from jax import lax
from jax.experimental import pallas as pl
from jax.experimental.pallas import tpu as pltpu
from jax.experimental.pallas import tpu_sc as plsc

TOTAL = 32768
D = 256
B = 16
DEGW = 2184
CH = 128
ACC_ROWS_PER_SEG = 16
NSEG_PER_SC = 8

_mesh = plsc.VectorSubcoreMesh(core_axis_name="c", subcore_axis_name="s")


def _body(x_hbm, nums_hbm, starts_hbm, degs_hbm, invs_hbm, out_hbm,
          deg_buf, xbuf, i2b, i4b, i8b, h0t, h1t, occ_tbl, nsv, stv, invv,
          zbuf, pbuf, sem_g, sem_s, acc):
    c = lax.axis_index("c")
    si = lax.axis_index("s")
    seg = c * NSEG_PER_SC + (si >> 1)
    half = si & 1
    seg_local = si >> 1
    base_row = seg_local * ACC_ROWS_PER_SEG
    lane = lax.iota(jnp.int32, 16)

    pltpu.sync_copy(nums_hbm, nsv)
    pltpu.sync_copy(starts_hbm, stv)
    L = jnp.sum(jnp.where(lane == seg, nsv[...], 0))
    start = jnp.sum(jnp.where(lane == seg, stv[...], 0))
    k2 = (L + 1) >> 1
    k4 = (L + 3) >> 2
    k8 = (L + 7) >> 3
    Lh0 = (L + 1) >> 1

    b0 = pl.multiple_of(jnp.minimum(start & (-8), TOTAL - DEGW), 8)
    off = start - b0
    pltpu.sync_copy(degs_hbm.at[pl.ds(b0, DEGW)], deg_buf)

    zero16i = jnp.zeros((16,), jnp.int32)
    for v in range(4):
        h0t[pl.ds(v * 16, 16)] = zero16i
        h1t[pl.ds(v * 16, 16)] = zero16i
    nv = (L + 15) >> 4

    def h_body(v, carry):
        d = deg_buf[pl.ds(off + v * 16, 16)]
        l_ = v * 16 + lane
        in0 = l_ < Lh0
        in1 = jnp.logical_and(l_ >= Lh0, l_ < L)
        cnt0, last0 = plsc.scan_count(d, in0)
        plsc.addupdate_scatter(h0t, [d], cnt0, mask=last0)
        cnt1, last1 = plsc.scan_count(d, in1)
        plsc.addupdate_scatter(h1t, [d], cnt1, mask=last1)
        return carry

    lax.fori_loop(0, nv, h_body, 0)

    h0 = [h0t[pl.ds(v * 16, 16)] for v in range(4)]
    h1 = [h1t[pl.ds(v * 16, 16)] for v in range(4)]
    ht = [a + b for a, b in zip(h0, h1)]
    t = [jnp.sum(hv) for hv in ht]
    above = [t[1] + t[2] + t[3], t[2] + t[3], t[3], jnp.int32(0)]
    for v in range(4):
        occ_tbl[pl.ds(v * 16, 16)] = (above[v] + (t[v] - plsc.cumsum(ht[v]))
                                      + half * h0[v])

    @pl.when(half == 0)
    def _():
        zero16f = jnp.zeros((16,), jnp.float32)
        for r in range(ACC_ROWS_PER_SEG):
            for v in range(D // 16):
                zbuf[r, pl.ds(v * 16, 16)] = zero16f
        pltpu.sync_copy(zbuf, acc.at[pl.ds(base_row, ACC_ROWS_PER_SEG)])

    plsc.subcore_barrier()

    lc0 = half * Lh0
    lim = Lh0 + half * (L - Lh0)
    nch = (lim - lc0 + CH - 1) >> 7
    trash = base_row + 15

    def c_body(ci, carry):
        gb = start + lc0 + ci * CH
        gbase = jnp.minimum(gb, TOTAL - CH)
        sh = gb - gbase
        gdesc = pltpu.async_copy(x_hbm.at[pl.ds(gbase, CH)], xbuf, sem_g)
        dstart = gbase - b0
        loc0 = gbase - start
        for sv in range(CH // 16):
            d = deg_buf[pl.ds(dstart + sv * 16, 16)]
            j = sv * 16 + lane
            l_ = loc0 + j
            valid = (j >= sh) & (l_ < lim)
            cnt, last = plsc.scan_count(d, valid)
            rk0 = plsc.load_gather(occ_tbl, [d])
            rank = rk0 + cnt - 1
            plsc.addupdate_scatter(occ_tbl, [d], cnt, mask=last)
            b2 = (rank >= k2).astype(jnp.int32)
            b4 = ((rank >= k4).astype(jnp.int32)
                  + (rank >= 2 * k4).astype(jnp.int32)
                  + (rank >= 3 * k4).astype(jnp.int32))
            b8 = (rank >= k8).astype(jnp.int32)
            for m in range(2, 8):
                b8 = b8 + (rank >= m * k8).astype(jnp.int32)
            i2b[pl.ds(sv * 16, 16)] = jnp.where(valid, base_row + b2, trash)
            i4b[pl.ds(sv * 16, 16)] = jnp.where(valid, base_row + 2 + b4, trash)
            i8b[pl.ds(sv * 16, 16)] = jnp.where(valid, base_row + 6 + b8, trash)
        gdesc.wait()
        s2 = pltpu.async_copy(xbuf, acc.at[i2b], sem_s, add=True)
        s4 = pltpu.async_copy(xbuf, acc.at[i4b], sem_s, add=True)
        s8 = pltpu.async_copy(xbuf, acc.at[i8b], sem_s, add=True)
        s2.wait()
        s4.wait()
        s8.wait()
        return carry

    lax.fori_loop(0, nch, c_body, 0)

    plsc.subcore_barrier()

    @pl.when(half == 0)
    def _():
        pltpu.sync_copy(acc.at[pl.ds(base_row, ACC_ROWS_PER_SEG)], zbuf)
        pltpu.sync_copy(invs_hbm, invv)
        zf = jnp.float32(0.0)
        invL = jnp.sum(jnp.where(lane == seg, invv[0, :], zf))
        inv2 = jnp.sum(jnp.where(lane == seg, invv[1, :], zf))
        inv4 = jnp.sum(jnp.where(lane == seg, invv[2, :], zf))
        inv8 = jnp.sum(jnp.where(lane == seg, invv[3, :], zf))
        for v in range(D // 16):
            sl = pl.ds(v * 16, 16)
            s2a = zbuf[0, sl]
            s2b = zbuf[1, sl]
            pbuf[0, sl] = (s2a + s2b) * invL
            pbuf[1, sl] = s2a * inv2
            pbuf[2, sl] = s2b * inv2
            for r in range(4):
                pbuf[3 + r, sl] = zbuf[2 + r, sl] * inv4
            for r in range(8):
                pbuf[7 + r, sl] = zbuf[6 + r, sl] * inv8
        pltpu.sync_copy(pbuf, out_hbm.at[pl.ds(seg * 15, 15)])


_pooling_kernel = functools.partial(
    pl.kernel,
    out_type=jax.ShapeDtypeStruct((B * 15, D), jnp.float32),
    mesh=_mesh,
    compiler_params=pltpu.CompilerParams(
        needs_layout_passes=False, use_tc_tiling_on_sc=False),
    scratch_types=[
        pltpu.VMEM((DEGW,), jnp.int32),
        pltpu.VMEM((CH, D), jnp.float32),
        pltpu.VMEM((CH,), jnp.int32),
        pltpu.VMEM((CH,), jnp.int32),
        pltpu.VMEM((CH,), jnp.int32),
        pltpu.VMEM((64,), jnp.int32),
        pltpu.VMEM((64,), jnp.int32),
        pltpu.VMEM((64,), jnp.int32),
        pltpu.VMEM((16,), jnp.int32),
        pltpu.VMEM((16,), jnp.int32),
        pltpu.VMEM((4, 16), jnp.float32),
        pltpu.VMEM((ACC_ROWS_PER_SEG, D), jnp.float32),
        pltpu.VMEM((15, D), jnp.float32),
        pltpu.SemaphoreType.DMA,
        pltpu.SemaphoreType.DMA,
        pltpu.VMEM_SHARED((NSEG_PER_SC * ACC_ROWS_PER_SEG, D), jnp.float32),
    ],
)(_body)


def _tr_body(src_ref, out_ref):
    src = src_ref[...]
    out_ref[...] = jnp.swapaxes(
        src.reshape(B, 15, D), 1, 2).reshape(B, D * 15)


_transpose_kernel = pl.pallas_call(
    _tr_body,
    out_shape=jax.ShapeDtypeStruct((B, D * 15), jnp.float32),
)


@jax.jit
def kernel(x, num_per_batch, degrees):
    nums = num_per_batch.astype(jnp.int32)
    starts = jnp.cumsum(nums) - nums
    lf = nums.astype(jnp.float32)
    invs = jnp.stack([
        1.0 / lf,
        1.0 / ((nums + 1) >> 1).astype(jnp.float32),
        1.0 / ((nums + 3) >> 2).astype(jnp.float32),
        1.0 / ((nums + 7) >> 3).astype(jnp.float32),
    ])
    pooled = _pooling_kernel(x, nums, starts, degrees.astype(jnp.int32), invs)
    return _transpose_kernel(pooled)


# speedup vs baseline: 1.0741x; 1.0741x over previous
import functools

import jax
import jax.numpy as jnp
from jax import lax
from jax.experimental import pallas as pl
from jax.experimental.pallas import tpu as pltpu
from jax.experimental.pallas import tpu_sc as plsc

TOTAL = 32768
D = 256
B = 16
NW = 32
RPW = TOTAL // NW
CH = 128
DEGW = 5248
NKEY = 1088
TRASH = 255

_mesh = plsc.VectorSubcoreMesh(core_axis_name="c", subcore_axis_name="s")


def _sc_body(nums_hbm, starts_hbm, degs_hbm, cols_hbm,
             deg_buf, c1b, c2b, c4b, c8b, occ_tbl, pre_tbl,
             nsv, stv, k2t, k4t, k8t):
    c = lax.axis_index("c")
    si = lax.axis_index("s")
    w = si * 2 + c
    lane = lax.iota(jnp.int32, 16)
    w_start = w * RPW

    pltpu.sync_copy(nums_hbm, nsv)
    pltpu.sync_copy(starts_hbm, stv)
    nums = nsv[...]
    starts_v = stv[...]
    cum_v = plsc.cumsum(nums)
    total_used = jnp.sum(jnp.where(lane == 15, cum_v, 0))

    one16 = jnp.full((16,), 1, jnp.int32)
    k2t[pl.ds(0, 16)] = (nums + 1) >> 1
    k2t[pl.ds(16, 16)] = one16
    k4t[pl.ds(0, 16)] = (nums + 3) >> 2
    k4t[pl.ds(16, 16)] = one16
    k8t[pl.ds(0, 16)] = (nums + 7) >> 3
    k8t[pl.ds(16, 16)] = one16

    cums = [jnp.sum(jnp.where(lane == s, cum_v, 0)) for s in range(16)]

    def seg_of(gpos):
        sv = (gpos >= cums[0]).astype(jnp.int32)
        for s in range(1, 16):
            sv = sv + (gpos >= cums[s]).astype(jnp.int32)
        return sv

    s0 = jnp.int32(0)
    s1 = jnp.int32(0)
    for s in range(16):
        s0 = s0 + (w_start >= cums[s]).astype(jnp.int32)
        s1 = s1 + (w_start + RPW - 1 >= cums[s]).astype(jnp.int32)
    A = jnp.sum(jnp.where(lane == s0, starts_v, 0))
    s1e = jnp.minimum(s1, 15)
    b_end = jnp.where(s0 >= 16, 0,
                      jnp.sum(jnp.where(lane == s1e, cum_v, 0)))

    b0 = pl.multiple_of(jnp.minimum(A & (-8), TOTAL - DEGW), 8)
    off = A - b0
    pltpu.sync_copy(degs_hbm.at[pl.ds(b0, DEGW)], deg_buf)

    zero16i = jnp.zeros((16,), jnp.int32)
    for v in range(NKEY // 16):
        occ_tbl[pl.ds(v * 16, 16)] = zero16i
        pre_tbl[pl.ds(v * 16, 16)] = zero16i

    nv = jnp.maximum(b_end - A + 15, 0) >> 4

    def h_body(v, carry):
        d = deg_buf[pl.ds(off + v * 16, 16)]
        gpos = A + v * 16 + lane
        key = seg_of(gpos) * 64 + d
        m_tot = gpos < b_end
        cnt, last = plsc.scan_count(key, m_tot)
        plsc.addupdate_scatter(occ_tbl, [key], cnt, mask=last)
        m_pre = gpos < w_start
        cntp, lastp = plsc.scan_count(key, m_pre)
        plsc.addupdate_scatter(pre_tbl, [key], cntp, mask=lastp)
        return carry

    lax.fori_loop(0, nv, h_body, 0)

    def s_body(s, carry):
        h = [occ_tbl[pl.ds(s * 64 + v * 16, 16)] for v in range(4)]
        t = [jnp.sum(hv) for hv in h]
        above = [t[1] + t[2] + t[3], t[2] + t[3], t[3], jnp.int32(0)]
        for v in range(4):
            sl = pl.ds(s * 64 + v * 16, 16)
            occ_tbl[sl] = above[v] + (t[v] - plsc.cumsum(h[v])) + pre_tbl[sl]
        return carry

    lax.fori_loop(s0, s1 + 1, s_body, 0)

    for ci in range(RPW // CH):
        gbase = w_start + ci * CH
        for sv in range(CH // 16):
            dsl = jnp.minimum(gbase - b0 + sv * 16, DEGW - 16)
            d = deg_buf[pl.ds(dsl, 16)]
            gpos = gbase + sv * 16 + lane
            segv = seg_of(gpos)
            valid = gpos < total_used
            key = segv * 64 + d
            cnt, last = plsc.scan_count(key, valid)
            rk0 = plsc.load_gather(occ_tbl, [key])
            rank = rk0 + cnt - 1
            plsc.addupdate_scatter(occ_tbl, [key], cnt, mask=last)
            k2 = plsc.load_gather(k2t, [segv])
            k4 = plsc.load_gather(k4t, [segv])
            k8 = plsc.load_gather(k8t, [segv])
            b2 = (rank >= k2).astype(jnp.int32)
            b4 = ((rank >= k4).astype(jnp.int32)
                  + (rank >= 2 * k4).astype(jnp.int32)
                  + (rank >= 3 * k4).astype(jnp.int32))
            b8 = (rank >= k8).astype(jnp.int32)
            for m in range(2, 8):
                b8 = b8 + (rank >= m * k8).astype(jnp.int32)
            cbase = segv * 15
            sl = pl.ds(sv * 16, 16)
            c1b[sl] = jnp.where(valid, cbase, TRASH)
            c2b[sl] = jnp.where(valid, cbase + 1 + b2, TRASH)
            c4b[sl] = jnp.where(valid, cbase + 3 + b4, TRASH)
            c8b[sl] = jnp.where(valid, cbase + 7 + b8, TRASH)
        pltpu.sync_copy(c1b, cols_hbm.at[0, pl.ds(gbase, CH)])
        pltpu.sync_copy(c2b, cols_hbm.at[1, pl.ds(gbase, CH)])
        pltpu.sync_copy(c4b, cols_hbm.at[2, pl.ds(gbase, CH)])
        pltpu.sync_copy(c8b, cols_hbm.at[3, pl.ds(gbase, CH)])


_sc_cols_kernel = functools.partial(
    pl.kernel,
    out_type=jax.ShapeDtypeStruct((4, TOTAL), jnp.int32),
    mesh=_mesh,
    compiler_params=pltpu.CompilerParams(
        needs_layout_passes=False, use_tc_tiling_on_sc=False),
    scratch_types=[
        pltpu.VMEM((DEGW,), jnp.int32),
        pltpu.VMEM((CH,), jnp.int32),
        pltpu.VMEM((CH,), jnp.int32),
        pltpu.VMEM((CH,), jnp.int32),
        pltpu.VMEM((CH,), jnp.int32),
        pltpu.VMEM((NKEY,), jnp.int32),
        pltpu.VMEM((NKEY,), jnp.int32),
        pltpu.VMEM((16,), jnp.int32),
        pltpu.VMEM((16,), jnp.int32),
        pltpu.VMEM((32,), jnp.int32),
        pltpu.VMEM((32,), jnp.int32),
        pltpu.VMEM((32,), jnp.int32),
    ],
)(_sc_body)

_RB = 512


def _acc_body(cols_ref, x_ref, o_ref):
    i = pl.program_id(0)

    @pl.when(i == 0)
    def _():
        o_ref[...] = jnp.zeros_like(o_ref)

    cols = cols_ref[...]
    xb = x_ref[...]
    cio = lax.broadcasted_iota(jnp.int32, (256, _RB), 0)
    wt = (cio == cols[0:1, :]).astype(jnp.float32)
    for l in range(1, 4):
        wt = wt + (cio == cols[l:l + 1, :]).astype(jnp.float32)
    o_ref[...] += lax.dot_general(
        wt, xb, (((1,), (0,)), ((), ())),
        preferred_element_type=jnp.float32)


_acc_kernel = pl.pallas_call(
    _acc_body,
    grid=(TOTAL // _RB,),
    in_specs=[
        pl.BlockSpec((4, _RB), lambda i: (0, i)),
        pl.BlockSpec((_RB, D), lambda i: (i, 0)),
    ],
    out_specs=pl.BlockSpec((256, D), lambda i: (0, 0)),
    out_shape=jax.ShapeDtypeStruct((256, D), jnp.float32),
)


def _tr_body(src_ref, inv_ref, out_ref):
    src = src_ref[...] * inv_ref[...]
    out_ref[...] = jnp.swapaxes(
        src[:B * 15].reshape(B, 15, D), 1, 2).reshape(B, D * 15)


_transpose_kernel = pl.pallas_call(
    _tr_body,
    out_shape=jax.ShapeDtypeStruct((B, D * 15), jnp.float32),
)


@jax.jit
def kernel(x, num_per_batch, degrees):
    nums = num_per_batch.astype(jnp.int32)
    starts = jnp.cumsum(nums) - nums
    lf = nums.astype(jnp.float32)
    invs = jnp.stack([
        1.0 / lf,
        1.0 / ((nums + 1) >> 1).astype(jnp.float32),
        1.0 / ((nums + 1) >> 1).astype(jnp.float32),
        1.0 / ((nums + 3) >> 2).astype(jnp.float32),
        1.0 / ((nums + 3) >> 2).astype(jnp.float32),
        1.0 / ((nums + 3) >> 2).astype(jnp.float32),
        1.0 / ((nums + 3) >> 2).astype(jnp.float32),
    ] + [1.0 / ((nums + 7) >> 3).astype(jnp.float32)] * 8, axis=1)
    invcol = jnp.zeros((256,), jnp.float32).at[:B * 15].set(invs.reshape(-1))
    invmat = jnp.broadcast_to(invcol[:, None], (256, D))
    cols = _sc_cols_kernel(nums, starts, degrees.astype(jnp.int32))
    acc = _acc_kernel(cols, x)
    return _transpose_kernel(acc, invmat)


# speedup vs baseline: 1.0918x; 1.0164x over previous
import functools

import jax
import jax.numpy as jnp
from jax import lax
from jax.experimental import pallas as pl
from jax.experimental.pallas import tpu as pltpu
from jax.experimental.pallas import tpu_sc as plsc

TOTAL = 32768
D = 256
B = 16
NW = 32
RPW = TOTAL // NW
CH = 128
DEGW = 5248
NKEY = 1088
TRASH = 255

_mesh = plsc.VectorSubcoreMesh(core_axis_name="c", subcore_axis_name="s")


def _sc_body(nums_hbm, starts_hbm, degs_hbm, cols_hbm,
             deg_buf, c1b, c2b, c4b, c8b, occ_tbl, pre_tbl,
             nsv, stv, k2t, k4t, k8t):
    c = lax.axis_index("c")
    si = lax.axis_index("s")
    w = si * 2 + c
    lane = lax.iota(jnp.int32, 16)
    w_start = w * RPW

    pltpu.sync_copy(nums_hbm, nsv)
    pltpu.sync_copy(starts_hbm, stv)
    nums = nsv[...]
    starts_v = stv[...]
    cum_v = plsc.cumsum(nums)
    total_used = jnp.sum(jnp.where(lane == 15, cum_v, 0))

    one16 = jnp.full((16,), 1, jnp.int32)
    k2t[pl.ds(0, 16)] = (nums + 1) >> 1
    k2t[pl.ds(16, 16)] = one16
    k4t[pl.ds(0, 16)] = (nums + 3) >> 2
    k4t[pl.ds(16, 16)] = one16
    k8t[pl.ds(0, 16)] = (nums + 7) >> 3
    k8t[pl.ds(16, 16)] = one16

    cums = [jnp.sum(jnp.where(lane == s, cum_v, 0)) for s in range(16)]

    def seg_of(gpos):
        sv = (gpos >= cums[0]).astype(jnp.int32)
        for s in range(1, 16):
            sv = sv + (gpos >= cums[s]).astype(jnp.int32)
        return sv

    s0 = jnp.int32(0)
    s1 = jnp.int32(0)
    for s in range(16):
        s0 = s0 + (w_start >= cums[s]).astype(jnp.int32)
        s1 = s1 + (w_start + RPW - 1 >= cums[s]).astype(jnp.int32)
    A = jnp.sum(jnp.where(lane == s0, starts_v, 0))
    s1e = jnp.minimum(s1, 15)
    b_end = jnp.where(s0 >= 16, 0,
                      jnp.sum(jnp.where(lane == s1e, cum_v, 0)))

    b0 = pl.multiple_of(jnp.minimum(A & (-8), TOTAL - DEGW), 8)
    off = A - b0
    pltpu.sync_copy(degs_hbm.at[pl.ds(b0, DEGW)], deg_buf)

    zero16i = jnp.zeros((16,), jnp.int32)
    for v in range(NKEY // 16):
        occ_tbl[pl.ds(v * 16, 16)] = zero16i
        pre_tbl[pl.ds(v * 16, 16)] = zero16i

    nv = jnp.maximum(b_end - A + 15, 0) >> 4

    def h_body(v, carry):
        d = deg_buf[pl.ds(off + v * 16, 16)]
        gpos = A + v * 16 + lane
        key = seg_of(gpos) * 64 + d
        m_tot = gpos < b_end
        cnt, last = plsc.scan_count(key, m_tot)
        plsc.addupdate_scatter(occ_tbl, [key], cnt, mask=last)
        m_pre = gpos < w_start
        cntp, lastp = plsc.scan_count(key, m_pre)
        plsc.addupdate_scatter(pre_tbl, [key], cntp, mask=lastp)
        return carry

    lax.fori_loop(0, nv, h_body, 0)

    def s_body(s, carry):
        h = [occ_tbl[pl.ds(s * 64 + v * 16, 16)] for v in range(4)]
        t = [jnp.sum(hv) for hv in h]
        above = [t[1] + t[2] + t[3], t[2] + t[3], t[3], jnp.int32(0)]
        for v in range(4):
            sl = pl.ds(s * 64 + v * 16, 16)
            occ_tbl[sl] = above[v] + (t[v] - plsc.cumsum(h[v])) + pre_tbl[sl]
        return carry

    lax.fori_loop(s0, s1 + 1, s_body, 0)

    for ci in range(RPW // CH):
        gbase = w_start + ci * CH
        for sv in range(CH // 16):
            dsl = jnp.minimum(gbase - b0 + sv * 16, DEGW - 16)
            d = deg_buf[pl.ds(dsl, 16)]
            gpos = gbase + sv * 16 + lane
            segv = seg_of(gpos)
            valid = gpos < total_used
            key = segv * 64 + d
            cnt, last = plsc.scan_count(key, valid)
            rk0 = plsc.load_gather(occ_tbl, [key])
            rank = rk0 + cnt - 1
            plsc.addupdate_scatter(occ_tbl, [key], cnt, mask=last)
            k2 = plsc.load_gather(k2t, [segv])
            k4 = plsc.load_gather(k4t, [segv])
            k8 = plsc.load_gather(k8t, [segv])
            b2 = (rank >= k2).astype(jnp.int32)
            b4 = ((rank >= k4).astype(jnp.int32)
                  + (rank >= 2 * k4).astype(jnp.int32)
                  + (rank >= 3 * k4).astype(jnp.int32))
            b8 = (rank >= k8).astype(jnp.int32)
            for m in range(2, 8):
                b8 = b8 + (rank >= m * k8).astype(jnp.int32)
            cbase = segv * 15
            sl = pl.ds(sv * 16, 16)
            c1b[sl] = jnp.where(valid, cbase, TRASH)
            c2b[sl] = jnp.where(valid, cbase + 1 + b2, TRASH)
            c4b[sl] = jnp.where(valid, cbase + 3 + b4, TRASH)
            c8b[sl] = jnp.where(valid, cbase + 7 + b8, TRASH)
        pltpu.sync_copy(c1b, cols_hbm.at[0, pl.ds(gbase, CH)])
        pltpu.sync_copy(c2b, cols_hbm.at[1, pl.ds(gbase, CH)])
        pltpu.sync_copy(c4b, cols_hbm.at[2, pl.ds(gbase, CH)])
        pltpu.sync_copy(c8b, cols_hbm.at[3, pl.ds(gbase, CH)])


_sc_cols_kernel = functools.partial(
    pl.kernel,
    out_type=jax.ShapeDtypeStruct((4, TOTAL), jnp.int32),
    mesh=_mesh,
    compiler_params=pltpu.CompilerParams(
        needs_layout_passes=False, use_tc_tiling_on_sc=False),
    scratch_types=[
        pltpu.VMEM((DEGW,), jnp.int32),
        pltpu.VMEM((CH,), jnp.int32),
        pltpu.VMEM((CH,), jnp.int32),
        pltpu.VMEM((CH,), jnp.int32),
        pltpu.VMEM((CH,), jnp.int32),
        pltpu.VMEM((NKEY,), jnp.int32),
        pltpu.VMEM((NKEY,), jnp.int32),
        pltpu.VMEM((16,), jnp.int32),
        pltpu.VMEM((16,), jnp.int32),
        pltpu.VMEM((32,), jnp.int32),
        pltpu.VMEM((32,), jnp.int32),
        pltpu.VMEM((32,), jnp.int32),
    ],
)(_sc_body)

_RB = 512


def _acc_body(cols_ref, x_ref, inv_ref, out_ref, acc_ref):
    i = pl.program_id(0)

    @pl.when(i == 0)
    def _():
        acc_ref[...] = jnp.zeros_like(acc_ref)

    cols = cols_ref[...]
    xb = x_ref[...]
    cio = lax.broadcasted_iota(jnp.int32, (256, _RB), 0)
    wt = (cio == cols[0:1, :]).astype(jnp.bfloat16)
    for l in range(1, 4):
        wt = wt + (cio == cols[l:l + 1, :]).astype(jnp.bfloat16)
    acc_ref[...] += lax.dot_general(
        wt, xb.astype(jnp.bfloat16), (((1,), (0,)), ((), ())),
        preferred_element_type=jnp.float32)

    @pl.when(i == TOTAL // _RB - 1)
    def _():
        src = acc_ref[...] * inv_ref[...]
        out_ref[...] = jnp.swapaxes(
            src[:B * 15].reshape(B, 15, D), 1, 2).reshape(B, D * 15)


_acc_kernel = pl.pallas_call(
    _acc_body,
    grid=(TOTAL // _RB,),
    in_specs=[
        pl.BlockSpec((4, _RB), lambda i: (0, i)),
        pl.BlockSpec((_RB, D), lambda i: (i, 0)),
        pl.BlockSpec((256, D), lambda i: (0, 0)),
    ],
    out_specs=pl.BlockSpec((B, D * 15), lambda i: (0, 0)),
    out_shape=jax.ShapeDtypeStruct((B, D * 15), jnp.float32),
    scratch_shapes=[pltpu.VMEM((256, D), jnp.float32)],
)


@jax.jit
def kernel(x, num_per_batch, degrees):
    nums = num_per_batch.astype(jnp.int32)
    starts = jnp.cumsum(nums) - nums
    lf = nums.astype(jnp.float32)
    invs = jnp.stack([
        1.0 / lf,
        1.0 / ((nums + 1) >> 1).astype(jnp.float32),
        1.0 / ((nums + 1) >> 1).astype(jnp.float32),
        1.0 / ((nums + 3) >> 2).astype(jnp.float32),
        1.0 / ((nums + 3) >> 2).astype(jnp.float32),
        1.0 / ((nums + 3) >> 2).astype(jnp.float32),
        1.0 / ((nums + 3) >> 2).astype(jnp.float32),
    ] + [1.0 / ((nums + 7) >> 3).astype(jnp.float32)] * 8, axis=1)
    invcol = jnp.zeros((256,), jnp.float32).at[:B * 15].set(invs.reshape(-1))
    invmat = jnp.broadcast_to(invcol[:, None], (256, D))
    cols = _sc_cols_kernel(nums, starts, degrees.astype(jnp.int32))
    return _acc_kernel(cols, x, invmat)


# speedup vs baseline: 1.4978x; 1.3719x over previous
import functools

import jax
import jax.numpy as jnp
from jax import lax
from jax.experimental import pallas as pl
from jax.experimental.pallas import tpu as pltpu
from jax.experimental.pallas import tpu_sc as plsc

TOTAL = 32768
D = 256
B = 16
NW = 32
RPW = TOTAL // NW
CH = 128
DEGW = 5248
NKEY = 1088
TRASH = 255

_mesh = plsc.VectorSubcoreMesh(core_axis_name="c", subcore_axis_name="s")


def _sc_body(nums_hbm, starts_hbm, degs_hbm, cols_hbm,
             deg_buf, c1b, c2b, c4b, c8b, occ_tbl, pre_tbl,
             nsv, stv, k2t, k4t, k8t):
    c = lax.axis_index("c")
    si = lax.axis_index("s")
    w = si * 2 + c
    lane = lax.iota(jnp.int32, 16)
    w_start = w * RPW

    pltpu.sync_copy(nums_hbm, nsv)
    pltpu.sync_copy(starts_hbm, stv)
    nums = nsv[...]
    starts_v = stv[...]
    cum_v = plsc.cumsum(nums)
    total_used = jnp.sum(jnp.where(lane == 15, cum_v, 0))

    one16 = jnp.full((16,), 1, jnp.int32)
    k2t[pl.ds(0, 16)] = (nums + 1) >> 1
    k2t[pl.ds(16, 16)] = one16
    k4t[pl.ds(0, 16)] = (nums + 3) >> 2
    k4t[pl.ds(16, 16)] = one16
    k8t[pl.ds(0, 16)] = (nums + 7) >> 3
    k8t[pl.ds(16, 16)] = one16

    cums = [jnp.sum(jnp.where(lane == s, cum_v, 0)) for s in range(16)]

    def seg_of(gpos):
        sv = (gpos >= cums[0]).astype(jnp.int32)
        for s in range(1, 16):
            sv = sv + (gpos >= cums[s]).astype(jnp.int32)
        return sv

    s0 = jnp.int32(0)
    s1 = jnp.int32(0)
    for s in range(16):
        s0 = s0 + (w_start >= cums[s]).astype(jnp.int32)
        s1 = s1 + (w_start + RPW - 1 >= cums[s]).astype(jnp.int32)
    A = jnp.sum(jnp.where(lane == s0, starts_v, 0))
    s1e = jnp.minimum(s1, 15)
    b_end = jnp.where(s0 >= 16, 0,
                      jnp.sum(jnp.where(lane == s1e, cum_v, 0)))

    b0 = pl.multiple_of(jnp.minimum(A & (-8), TOTAL - DEGW), 8)
    off = A - b0
    pltpu.sync_copy(degs_hbm.at[pl.ds(b0, DEGW)], deg_buf)

    zero16i = jnp.zeros((16,), jnp.int32)
    for v in range(NKEY // 16):
        occ_tbl[pl.ds(v * 16, 16)] = zero16i
        pre_tbl[pl.ds(v * 16, 16)] = zero16i

    nv = jnp.maximum(b_end - A + 15, 0) >> 4

    def h_body(v, carry):
        d = deg_buf[pl.ds(off + v * 16, 16)]
        gpos = A + v * 16 + lane
        key = seg_of(gpos) * 64 + d
        m_tot = gpos < b_end
        cnt, last = plsc.scan_count(key, m_tot)
        plsc.addupdate_scatter(occ_tbl, [key], cnt, mask=last)
        m_pre = gpos < w_start
        cntp, lastp = plsc.scan_count(key, m_pre)
        plsc.addupdate_scatter(pre_tbl, [key], cntp, mask=lastp)
        return carry

    lax.fori_loop(0, nv, h_body, 0)

    def s_body(s, carry):
        h = [occ_tbl[pl.ds(s * 64 + v * 16, 16)] for v in range(4)]
        t = [jnp.sum(hv) for hv in h]
        above = [t[1] + t[2] + t[3], t[2] + t[3], t[3], jnp.int32(0)]
        for v in range(4):
            sl = pl.ds(s * 64 + v * 16, 16)
            occ_tbl[sl] = above[v] + (t[v] - plsc.cumsum(h[v])) + pre_tbl[sl]
        return carry

    lax.fori_loop(s0, s1 + 1, s_body, 0)

    def c_body(ci, carry):
        gbase = pl.multiple_of(w_start + ci * CH, CH)
        for sv in range(CH // 16):
            dsl = jnp.minimum(gbase - b0 + sv * 16, DEGW - 16)
            d = deg_buf[pl.ds(dsl, 16)]
            gpos = gbase + sv * 16 + lane
            segv = seg_of(gpos)
            valid = gpos < total_used
            key = segv * 64 + d
            cnt, last = plsc.scan_count(key, valid)
            rk0 = plsc.load_gather(occ_tbl, [key])
            rank = rk0 + cnt - 1
            plsc.addupdate_scatter(occ_tbl, [key], cnt, mask=last)
            k2 = plsc.load_gather(k2t, [segv])
            k4 = plsc.load_gather(k4t, [segv])
            k8 = plsc.load_gather(k8t, [segv])
            b2 = (rank >= k2).astype(jnp.int32)
            b4 = ((rank >= k4).astype(jnp.int32)
                  + (rank >= 2 * k4).astype(jnp.int32)
                  + (rank >= 3 * k4).astype(jnp.int32))
            b8 = (rank >= k8).astype(jnp.int32)
            for m in range(2, 8):
                b8 = b8 + (rank >= m * k8).astype(jnp.int32)
            cbase = segv * 15
            sl = pl.ds(sv * 16, 16)
            c1b[sl] = jnp.where(valid, cbase, TRASH)
            c2b[sl] = jnp.where(valid, cbase + 1 + b2, TRASH)
            c4b[sl] = jnp.where(valid, cbase + 3 + b4, TRASH)
            c8b[sl] = jnp.where(valid, cbase + 7 + b8, TRASH)
        pltpu.sync_copy(c1b, cols_hbm.at[0, pl.ds(gbase, CH)])
        pltpu.sync_copy(c2b, cols_hbm.at[1, pl.ds(gbase, CH)])
        pltpu.sync_copy(c4b, cols_hbm.at[2, pl.ds(gbase, CH)])
        pltpu.sync_copy(c8b, cols_hbm.at[3, pl.ds(gbase, CH)])
        return carry

    lax.fori_loop(0, RPW // CH, c_body, 0)


_sc_cols_kernel = functools.partial(
    pl.kernel,
    out_type=jax.ShapeDtypeStruct((4, TOTAL), jnp.int32),
    mesh=_mesh,
    compiler_params=pltpu.CompilerParams(
        needs_layout_passes=False, use_tc_tiling_on_sc=False),
    scratch_types=[
        pltpu.VMEM((DEGW,), jnp.int32),
        pltpu.VMEM((CH,), jnp.int32),
        pltpu.VMEM((CH,), jnp.int32),
        pltpu.VMEM((CH,), jnp.int32),
        pltpu.VMEM((CH,), jnp.int32),
        pltpu.VMEM((NKEY,), jnp.int32),
        pltpu.VMEM((NKEY,), jnp.int32),
        pltpu.VMEM((16,), jnp.int32),
        pltpu.VMEM((16,), jnp.int32),
        pltpu.VMEM((32,), jnp.int32),
        pltpu.VMEM((32,), jnp.int32),
        pltpu.VMEM((32,), jnp.int32),
    ],
)(_sc_body)

_RB = 1024


def _acc_body(cols_ref, x_ref, inv_ref, out_ref, acc_ref):
    i = pl.program_id(0)

    @pl.when(i == 0)
    def _():
        acc_ref[...] = jnp.zeros_like(acc_ref)

    cols = cols_ref[...]
    xb = x_ref[...]
    cio = lax.broadcasted_iota(jnp.int32, (256, _RB), 0)
    wt = (cio == cols[0:1, :]).astype(jnp.bfloat16)
    for l in range(1, 4):
        wt = wt + (cio == cols[l:l + 1, :]).astype(jnp.bfloat16)
    acc_ref[...] += lax.dot_general(
        wt, xb.astype(jnp.bfloat16), (((1,), (0,)), ((), ())),
        preferred_element_type=jnp.float32)

    @pl.when(i == TOTAL // _RB - 1)
    def _():
        src = acc_ref[...] * inv_ref[...]
        out_ref[...] = jnp.swapaxes(
            src[:B * 15].reshape(B, 15, D), 1, 2).reshape(B, D * 15)


_acc_kernel = pl.pallas_call(
    _acc_body,
    grid=(TOTAL // _RB,),
    in_specs=[
        pl.BlockSpec((4, _RB), lambda i: (0, i)),
        pl.BlockSpec((_RB, D), lambda i: (i, 0)),
        pl.BlockSpec((256, D), lambda i: (0, 0)),
    ],
    out_specs=pl.BlockSpec((B, D * 15), lambda i: (0, 0)),
    out_shape=jax.ShapeDtypeStruct((B, D * 15), jnp.float32),
    scratch_shapes=[pltpu.VMEM((256, D), jnp.float32)],
)


@jax.jit
def kernel(x, num_per_batch, degrees):
    nums = num_per_batch.astype(jnp.int32)
    starts = jnp.cumsum(nums) - nums
    lf = nums.astype(jnp.float32)
    invs = jnp.stack([
        1.0 / lf,
        1.0 / ((nums + 1) >> 1).astype(jnp.float32),
        1.0 / ((nums + 1) >> 1).astype(jnp.float32),
        1.0 / ((nums + 3) >> 2).astype(jnp.float32),
        1.0 / ((nums + 3) >> 2).astype(jnp.float32),
        1.0 / ((nums + 3) >> 2).astype(jnp.float32),
        1.0 / ((nums + 3) >> 2).astype(jnp.float32),
    ] + [1.0 / ((nums + 7) >> 3).astype(jnp.float32)] * 8, axis=1)
    invcol = jnp.zeros((256,), jnp.float32).at[:B * 15].set(invs.reshape(-1))
    invmat = jnp.broadcast_to(invcol[:, None], (256, D))
    cols = _sc_cols_kernel(nums, starts, degrees.astype(jnp.int32))
    return _acc_kernel(cols, x, invmat)


# speedup vs baseline: 1.7219x; 1.1496x over previous
import functools

import jax
import jax.numpy as jnp
from jax import lax
from jax.experimental import pallas as pl
from jax.experimental.pallas import tpu as pltpu
from jax.experimental.pallas import tpu_sc as plsc

TOTAL = 32768
D = 256
B = 16
NW = 32
RPW = TOTAL // NW
CH = 128
DEGW = 5248
NKEY = 1088
TRASH = 255

_mesh = plsc.VectorSubcoreMesh(core_axis_name="c", subcore_axis_name="s")


def _sc_body(nums_hbm, starts_hbm, degs_hbm, cols_hbm,
             deg_buf, c1b, c2b, c4b, c8b, occ_tbl, pre_tbl,
             nsv, stv, k2t, k4t, k8t):
    c = lax.axis_index("c")
    si = lax.axis_index("s")
    w = si * 2 + c
    lane = lax.iota(jnp.int32, 16)
    w_start = w * RPW

    pltpu.sync_copy(nums_hbm, nsv)
    pltpu.sync_copy(starts_hbm, stv)
    nums = nsv[...]
    starts_v = stv[...]
    cum_v = plsc.cumsum(nums)
    total_used = jnp.sum(jnp.where(lane == 15, cum_v, 0))

    one16 = jnp.full((16,), 1, jnp.int32)
    k2t[pl.ds(0, 16)] = (nums + 1) >> 1
    k2t[pl.ds(16, 16)] = one16
    k4t[pl.ds(0, 16)] = (nums + 3) >> 2
    k4t[pl.ds(16, 16)] = one16
    k8t[pl.ds(0, 16)] = (nums + 7) >> 3
    k8t[pl.ds(16, 16)] = one16

    cums = [jnp.sum(jnp.where(lane == s, cum_v, 0)) for s in range(16)]

    def seg_of(gpos):
        sv = (gpos >= cums[0]).astype(jnp.int32)
        for s in range(1, 16):
            sv = sv + (gpos >= cums[s]).astype(jnp.int32)
        return sv

    s0 = jnp.int32(0)
    s1 = jnp.int32(0)
    for s in range(16):
        s0 = s0 + (w_start >= cums[s]).astype(jnp.int32)
        s1 = s1 + (w_start + RPW - 1 >= cums[s]).astype(jnp.int32)
    A = jnp.sum(jnp.where(lane == s0, starts_v, 0))
    s1e = jnp.minimum(s1, 15)
    b_end = jnp.where(s0 >= 16, 0,
                      jnp.sum(jnp.where(lane == s1e, cum_v, 0)))

    b0 = pl.multiple_of(jnp.minimum(A & (-8), TOTAL - DEGW), 8)
    off = A - b0
    pltpu.sync_copy(degs_hbm.at[pl.ds(b0, DEGW)], deg_buf)

    zero16i = jnp.zeros((16,), jnp.int32)
    for v in range(NKEY // 16):
        occ_tbl[pl.ds(v * 16, 16)] = zero16i
        pre_tbl[pl.ds(v * 16, 16)] = zero16i

    nv = jnp.maximum(b_end - A + 15, 0) >> 4

    def h_body(v, carry):
        d = deg_buf[pl.ds(off + v * 16, 16)]
        gpos = A + v * 16 + lane
        key = seg_of(gpos) * 64 + d
        m_tot = gpos < b_end
        cnt, last = plsc.scan_count(key, m_tot)
        plsc.addupdate_scatter(occ_tbl, [key], cnt, mask=last)
        m_pre = gpos < w_start
        cntp, lastp = plsc.scan_count(key, m_pre)
        plsc.addupdate_scatter(pre_tbl, [key], cntp, mask=lastp)
        return carry

    lax.fori_loop(0, nv, h_body, 0)

    def s_body(s, carry):
        h = [occ_tbl[pl.ds(s * 64 + v * 16, 16)] for v in range(4)]
        t = [jnp.sum(hv) for hv in h]
        above = [t[1] + t[2] + t[3], t[2] + t[3], t[3], jnp.int32(0)]
        for v in range(4):
            sl = pl.ds(s * 64 + v * 16, 16)
            occ_tbl[sl] = above[v] + (t[v] - plsc.cumsum(h[v])) + pre_tbl[sl]
        return carry

    lax.fori_loop(s0, s1 + 1, s_body, 0)

    def c_body(ci, carry):
        gbase = pl.multiple_of(w_start + ci * CH, CH)
        for sv in range(CH // 16):
            dsl = jnp.minimum(gbase - b0 + sv * 16, DEGW - 16)
            d = deg_buf[pl.ds(dsl, 16)]
            gpos = gbase + sv * 16 + lane
            segv = seg_of(gpos)
            valid = gpos < total_used
            key = segv * 64 + d
            cnt, last = plsc.scan_count(key, valid)
            rk0 = plsc.load_gather(occ_tbl, [key])
            rank = rk0 + cnt - 1
            plsc.addupdate_scatter(occ_tbl, [key], cnt, mask=last)
            k2 = plsc.load_gather(k2t, [segv])
            k4 = plsc.load_gather(k4t, [segv])
            k8 = plsc.load_gather(k8t, [segv])
            b2 = (rank >= k2).astype(jnp.int32)
            b4 = ((rank >= k4).astype(jnp.int32)
                  + (rank >= 2 * k4).astype(jnp.int32)
                  + (rank >= 3 * k4).astype(jnp.int32))
            b8 = (rank >= k8).astype(jnp.int32)
            for m in range(2, 8):
                b8 = b8 + (rank >= m * k8).astype(jnp.int32)
            cbase = segv * 15
            sl = pl.ds(sv * 16, 16)
            c1b[sl] = jnp.where(valid, cbase, TRASH)
            c2b[sl] = jnp.where(valid, cbase + 1 + b2, TRASH)
            c4b[sl] = jnp.where(valid, cbase + 3 + b4, TRASH)
            c8b[sl] = jnp.where(valid, cbase + 7 + b8, TRASH)
        pltpu.sync_copy(c1b, cols_hbm.at[0, pl.ds(gbase, CH)])
        pltpu.sync_copy(c2b, cols_hbm.at[1, pl.ds(gbase, CH)])
        pltpu.sync_copy(c4b, cols_hbm.at[2, pl.ds(gbase, CH)])
        pltpu.sync_copy(c8b, cols_hbm.at[3, pl.ds(gbase, CH)])
        return carry

    lax.fori_loop(0, RPW // CH, c_body, 0)


_sc_cols_kernel = functools.partial(
    pl.kernel,
    out_type=jax.ShapeDtypeStruct((4, TOTAL), jnp.int32),
    mesh=_mesh,
    compiler_params=pltpu.CompilerParams(
        needs_layout_passes=False, use_tc_tiling_on_sc=False),
    scratch_types=[
        pltpu.VMEM((DEGW,), jnp.int32),
        pltpu.VMEM((CH,), jnp.int32),
        pltpu.VMEM((CH,), jnp.int32),
        pltpu.VMEM((CH,), jnp.int32),
        pltpu.VMEM((CH,), jnp.int32),
        pltpu.VMEM((NKEY,), jnp.int32),
        pltpu.VMEM((NKEY,), jnp.int32),
        pltpu.VMEM((16,), jnp.int32),
        pltpu.VMEM((16,), jnp.int32),
        pltpu.VMEM((32,), jnp.int32),
        pltpu.VMEM((32,), jnp.int32),
        pltpu.VMEM((32,), jnp.int32),
    ],
)(_sc_body)

_RB = 2048


def _acc_body(cols_ref, x_ref, inv_ref, out_ref, acc_ref):
    i = pl.program_id(0)

    @pl.when(i == 0)
    def _():
        acc_ref[...] = jnp.zeros_like(acc_ref)

    cols = cols_ref[...]
    xb = x_ref[...]
    cio = lax.broadcasted_iota(jnp.int32, (256, _RB), 0)
    wt = (cio == cols[0:1, :]).astype(jnp.bfloat16)
    for l in range(1, 4):
        wt = wt + (cio == cols[l:l + 1, :]).astype(jnp.bfloat16)
    acc_ref[...] += lax.dot_general(
        wt, xb.astype(jnp.bfloat16), (((1,), (0,)), ((), ())),
        preferred_element_type=jnp.float32)

    @pl.when(i == TOTAL // _RB - 1)
    def _():
        src = acc_ref[...] * inv_ref[...]
        out_ref[...] = jnp.swapaxes(
            src[:B * 15].reshape(B, 15, D), 1, 2).reshape(B, D * 15)


_acc_kernel = pl.pallas_call(
    _acc_body,
    grid=(TOTAL // _RB,),
    in_specs=[
        pl.BlockSpec((4, _RB), lambda i: (0, i)),
        pl.BlockSpec((_RB, D), lambda i: (i, 0)),
        pl.BlockSpec((256, D), lambda i: (0, 0)),
    ],
    out_specs=pl.BlockSpec((B, D * 15), lambda i: (0, 0)),
    out_shape=jax.ShapeDtypeStruct((B, D * 15), jnp.float32),
    scratch_shapes=[pltpu.VMEM((256, D), jnp.float32)],
)


@jax.jit
def kernel(x, num_per_batch, degrees):
    nums = num_per_batch.astype(jnp.int32)
    starts = jnp.cumsum(nums) - nums
    lf = nums.astype(jnp.float32)
    invs = jnp.stack([
        1.0 / lf,
        1.0 / ((nums + 1) >> 1).astype(jnp.float32),
        1.0 / ((nums + 1) >> 1).astype(jnp.float32),
        1.0 / ((nums + 3) >> 2).astype(jnp.float32),
        1.0 / ((nums + 3) >> 2).astype(jnp.float32),
        1.0 / ((nums + 3) >> 2).astype(jnp.float32),
        1.0 / ((nums + 3) >> 2).astype(jnp.float32),
    ] + [1.0 / ((nums + 7) >> 3).astype(jnp.float32)] * 8, axis=1)
    invcol = jnp.zeros((256,), jnp.float32).at[:B * 15].set(invs.reshape(-1))
    invmat = jnp.broadcast_to(invcol[:, None], (256, D))
    cols = _sc_cols_kernel(nums, starts, degrees.astype(jnp.int32))
    return _acc_kernel(cols, x, invmat)


# speedup vs baseline: 1.8273x; 1.0612x over previous
import functools

import jax
import jax.numpy as jnp
from jax import lax
from jax.experimental import pallas as pl
from jax.experimental.pallas import tpu as pltpu
from jax.experimental.pallas import tpu_sc as plsc

TOTAL = 32768
D = 256
B = 16
NW = 32
RPW = TOTAL // NW
CH = 128
DEGW = 5248
NKEY = 1088
TRASH = 255

_mesh = plsc.VectorSubcoreMesh(core_axis_name="c", subcore_axis_name="s")


def _sc_body(nums_hbm, starts_hbm, degs_hbm, cols_hbm,
             deg_buf, c1b, c2b, c4b, c8b, occ_tbl, pre_tbl,
             nsv, stv, k2t, k4t, k8t):
    c = lax.axis_index("c")
    si = lax.axis_index("s")
    w = si * 2 + c
    lane = lax.iota(jnp.int32, 16)
    w_start = w * RPW

    pltpu.sync_copy(nums_hbm, nsv)
    pltpu.sync_copy(starts_hbm, stv)
    nums = nsv[...]
    starts_v = stv[...]
    cum_v = plsc.cumsum(nums)
    total_used = jnp.sum(jnp.where(lane == 15, cum_v, 0))

    one16 = jnp.full((16,), 1, jnp.int32)
    k2t[pl.ds(0, 16)] = (nums + 1) >> 1
    k2t[pl.ds(16, 16)] = one16
    k4t[pl.ds(0, 16)] = (nums + 3) >> 2
    k4t[pl.ds(16, 16)] = one16
    k8t[pl.ds(0, 16)] = (nums + 7) >> 3
    k8t[pl.ds(16, 16)] = one16

    cums = [jnp.sum(jnp.where(lane == s, cum_v, 0)) for s in range(16)]

    def seg_of(gpos):
        sv = (gpos >= cums[0]).astype(jnp.int32)
        for s in range(1, 16):
            sv = sv + (gpos >= cums[s]).astype(jnp.int32)
        return sv

    s0 = jnp.int32(0)
    s1 = jnp.int32(0)
    for s in range(16):
        s0 = s0 + (w_start >= cums[s]).astype(jnp.int32)
        s1 = s1 + (w_start + RPW - 1 >= cums[s]).astype(jnp.int32)
    A = jnp.sum(jnp.where(lane == s0, starts_v, 0))
    s1e = jnp.minimum(s1, 15)
    b_end = jnp.where(s0 >= 16, 0,
                      jnp.sum(jnp.where(lane == s1e, cum_v, 0)))

    b0 = pl.multiple_of(jnp.minimum(A & (-8), TOTAL - DEGW), 8)
    off = A - b0
    pltpu.sync_copy(degs_hbm.at[pl.ds(b0, DEGW)], deg_buf)

    zero16i = jnp.zeros((16,), jnp.int32)
    for v in range(NKEY // 16):
        occ_tbl[pl.ds(v * 16, 16)] = zero16i
        pre_tbl[pl.ds(v * 16, 16)] = zero16i

    nv = jnp.maximum(b_end - A + 15, 0) >> 4

    def h_body(v, carry):
        d = deg_buf[pl.ds(off + v * 16, 16)]
        gpos = A + v * 16 + lane
        key = seg_of(gpos) * 64 + d
        m_tot = gpos < b_end
        cnt, last = plsc.scan_count(key, m_tot)
        plsc.addupdate_scatter(occ_tbl, [key], cnt, mask=last)
        m_pre = gpos < w_start
        cntp, lastp = plsc.scan_count(key, m_pre)
        plsc.addupdate_scatter(pre_tbl, [key], cntp, mask=lastp)
        return carry

    lax.fori_loop(0, nv, h_body, 0)

    def s_body(s, carry):
        h = [occ_tbl[pl.ds(s * 64 + v * 16, 16)] for v in range(4)]
        t = [jnp.sum(hv) for hv in h]
        above = [t[1] + t[2] + t[3], t[2] + t[3], t[3], jnp.int32(0)]
        for v in range(4):
            sl = pl.ds(s * 64 + v * 16, 16)
            occ_tbl[sl] = above[v] + (t[v] - plsc.cumsum(h[v])) + pre_tbl[sl]
        return carry

    lax.fori_loop(s0, s1 + 1, s_body, 0)

    def c_body(ci, carry):
        gbase = pl.multiple_of(w_start + ci * CH, CH)
        for sv in range(CH // 16):
            dsl = jnp.minimum(gbase - b0 + sv * 16, DEGW - 16)
            d = deg_buf[pl.ds(dsl, 16)]
            gpos = gbase + sv * 16 + lane
            segv = seg_of(gpos)
            valid = gpos < total_used
            key = segv * 64 + d
            cnt, last = plsc.scan_count(key, valid)
            rk0 = plsc.load_gather(occ_tbl, [key])
            rank = rk0 + cnt - 1
            plsc.addupdate_scatter(occ_tbl, [key], cnt, mask=last)
            k2 = plsc.load_gather(k2t, [segv])
            k4 = plsc.load_gather(k4t, [segv])
            k8 = plsc.load_gather(k8t, [segv])
            b2 = (rank >= k2).astype(jnp.int32)
            b4 = ((rank >= k4).astype(jnp.int32)
                  + (rank >= 2 * k4).astype(jnp.int32)
                  + (rank >= 3 * k4).astype(jnp.int32))
            b8 = (rank >= k8).astype(jnp.int32)
            for m in range(2, 8):
                b8 = b8 + (rank >= m * k8).astype(jnp.int32)
            cbase = segv * 15
            sl = pl.ds(sv * 16, 16)
            c1b[sl] = jnp.where(valid, cbase, TRASH)
            c2b[sl] = jnp.where(valid, cbase + 1 + b2, TRASH)
            c4b[sl] = jnp.where(valid, cbase + 3 + b4, TRASH)
            c8b[sl] = jnp.where(valid, cbase + 7 + b8, TRASH)
        pltpu.sync_copy(c1b, cols_hbm.at[0, pl.ds(gbase, CH)])
        pltpu.sync_copy(c2b, cols_hbm.at[1, pl.ds(gbase, CH)])
        pltpu.sync_copy(c4b, cols_hbm.at[2, pl.ds(gbase, CH)])
        pltpu.sync_copy(c8b, cols_hbm.at[3, pl.ds(gbase, CH)])
        return carry

    lax.fori_loop(0, RPW // CH, c_body, 0)


_sc_cols_kernel = functools.partial(
    pl.kernel,
    out_type=jax.ShapeDtypeStruct((4, TOTAL), jnp.int32),
    mesh=_mesh,
    compiler_params=pltpu.CompilerParams(
        needs_layout_passes=False, use_tc_tiling_on_sc=False),
    scratch_types=[
        pltpu.VMEM((DEGW,), jnp.int32),
        pltpu.VMEM((CH,), jnp.int32),
        pltpu.VMEM((CH,), jnp.int32),
        pltpu.VMEM((CH,), jnp.int32),
        pltpu.VMEM((CH,), jnp.int32),
        pltpu.VMEM((NKEY,), jnp.int32),
        pltpu.VMEM((NKEY,), jnp.int32),
        pltpu.VMEM((16,), jnp.int32),
        pltpu.VMEM((16,), jnp.int32),
        pltpu.VMEM((32,), jnp.int32),
        pltpu.VMEM((32,), jnp.int32),
        pltpu.VMEM((32,), jnp.int32),
    ],
)(_sc_body)

_RB = 4096


def _acc_body(cols_ref, x_ref, inv_ref, out_ref, acc_ref):
    i = pl.program_id(0)

    @pl.when(i == 0)
    def _():
        acc_ref[...] = jnp.zeros_like(acc_ref)

    cols = cols_ref[...]
    xb = x_ref[...]
    cio = lax.broadcasted_iota(jnp.int32, (256, _RB), 0)
    wt = (cio == cols[0:1, :]).astype(jnp.bfloat16)
    for l in range(1, 4):
        wt = wt + (cio == cols[l:l + 1, :]).astype(jnp.bfloat16)
    acc_ref[...] += lax.dot_general(
        wt, xb.astype(jnp.bfloat16), (((1,), (0,)), ((), ())),
        preferred_element_type=jnp.float32)

    @pl.when(i == TOTAL // _RB - 1)
    def _():
        src = acc_ref[...] * inv_ref[...]
        out_ref[...] = jnp.swapaxes(
            src[:B * 15].reshape(B, 15, D), 1, 2).reshape(B, D * 15)


_acc_kernel = pl.pallas_call(
    _acc_body,
    grid=(TOTAL // _RB,),
    in_specs=[
        pl.BlockSpec((4, _RB), lambda i: (0, i)),
        pl.BlockSpec((_RB, D), lambda i: (i, 0)),
        pl.BlockSpec((256, D), lambda i: (0, 0)),
    ],
    out_specs=pl.BlockSpec((B, D * 15), lambda i: (0, 0)),
    out_shape=jax.ShapeDtypeStruct((B, D * 15), jnp.float32),
    scratch_shapes=[pltpu.VMEM((256, D), jnp.float32)],
)


@jax.jit
def kernel(x, num_per_batch, degrees):
    nums = num_per_batch.astype(jnp.int32)
    starts = jnp.cumsum(nums) - nums
    lf = nums.astype(jnp.float32)
    invs = jnp.stack([
        1.0 / lf,
        1.0 / ((nums + 1) >> 1).astype(jnp.float32),
        1.0 / ((nums + 1) >> 1).astype(jnp.float32),
        1.0 / ((nums + 3) >> 2).astype(jnp.float32),
        1.0 / ((nums + 3) >> 2).astype(jnp.float32),
        1.0 / ((nums + 3) >> 2).astype(jnp.float32),
        1.0 / ((nums + 3) >> 2).astype(jnp.float32),
    ] + [1.0 / ((nums + 7) >> 3).astype(jnp.float32)] * 8, axis=1)
    invcol = jnp.zeros((256,), jnp.float32).at[:B * 15].set(invs.reshape(-1))
    invmat = jnp.broadcast_to(invcol[:, None], (256, D))
    cols = _sc_cols_kernel(nums, starts, degrees.astype(jnp.int32))
    return _acc_kernel(cols, x, invmat)
